# baseline (speedup 1.0000x reference)
import functools

import jax
import jax.numpy as jnp
from jax import lax
from jax.experimental import pallas as pl
from jax.experimental.pallas import tpu as pltpu
from jax.experimental.pallas import tpu_sc as plsc

N = 16384
C = 100
L = 16
NV = 7
NW = 32
RPW = N // NW
CH = 128
NCH = RPW // CH
PAD = -1.0


def _sa(v):
    return plsc.sort_key_val(v, v)[0]


def _sd(v):
    return plsc.sort_key_val(v, v, descending=True)[0]


def _sort_top112(p, w6, padv):
    a0, d1 = _sa(p[0]), _sd(p[1])
    d2, a3 = _sd(p[2]), _sa(p[3])
    d4, a5 = _sd(p[4]), _sa(p[5])
    a6 = _sa(w6)
    lo, hi = jnp.minimum(a0, d1), jnp.maximum(a0, d1)
    b01 = [_sa(lo), _sa(hi)]
    lo, hi = jnp.minimum(d2, a3), jnp.maximum(d2, a3)
    b23 = [_sd(hi), _sd(lo)]
    lo, hi = jnp.minimum(d4, a5), jnp.maximum(d4, a5)
    b45 = [_sd(hi), _sd(lo)]
    l0, l1 = jnp.minimum(b01[0], b23[0]), jnp.minimum(b01[1], b23[1])
    h0, h1 = jnp.maximum(b01[0], b23[0]), jnp.maximum(b01[1], b23[1])
    q0 = [
        _sa(jnp.minimum(l0, l1)), _sa(jnp.maximum(l0, l1)),
        _sa(jnp.minimum(h0, h1)), _sa(jnp.maximum(h0, h1)),
    ]
    m1, mm1 = jnp.maximum(b45[1], a6), jnp.minimum(b45[1], a6)
    q1 = [
        _sd(jnp.maximum(b45[0], m1)), _sd(jnp.minimum(b45[0], m1)),
        _sd(mm1), padv,
    ]
    f = [jnp.minimum(q0[i], q1[i]) for i in range(3)]
    g = [jnp.maximum(q0[i], q1[i]) for i in range(3)] + [q0[3]]
    gm0, gm1 = jnp.minimum(g[0], g[2]), jnp.minimum(g[1], g[3])
    gM0, gM1 = jnp.maximum(g[0], g[2]), jnp.maximum(g[1], g[3])
    s4, s5 = _sa(jnp.minimum(gm0, gm1)), _sa(jnp.maximum(gm0, gm1))
    s6, s7 = _sa(jnp.minimum(gM0, gM1)), _sa(jnp.maximum(gM0, gM1))
    mm, MM = jnp.minimum(f[0], f[2]), jnp.maximum(f[0], f[2])
    s1 = _sa(mm)
    s2, s3 = _sa(jnp.minimum(MM, f[1])), _sa(jnp.maximum(MM, f[1]))
    return [s1, s2, s3, s4, s5, s6, s7]


def _make_meter_kernel():
    mesh = plsc.VectorSubcoreMesh(core_axis_name="c", subcore_axis_name="s")

    @functools.partial(
        pl.kernel,
        out_type=(
            jax.ShapeDtypeStruct((NW, NV, L), jnp.float32),
            jax.ShapeDtypeStruct((NW, NV, L), jnp.float32),
        ),
        mesh=mesh,
        compiler_params=pltpu.CompilerParams(
            needs_layout_passes=False, use_tc_tiling_on_sc=False),
        scratch_types=[
            pltpu.VMEM((CH, C), jnp.float32),
            pltpu.VMEM((CH, C), jnp.float32),
            pltpu.VMEM((CH, C), jnp.float32),
            pltpu.VMEM((CH, C), jnp.float32),
            pltpu.VMEM((NV, L), jnp.float32),
            pltpu.VMEM((NV, L), jnp.float32),
            pltpu.VMEM((8 * 8 * L,), jnp.float32),
            pltpu.SemaphoreType.DMA,
            pltpu.SemaphoreType.DMA,
        ],
    )
    def meter(preds_hbm, targets_hbm, corr_out, tot_out,
              pv0, tv0, pv1, tv1, cst, tst, sortbuf, sem0, sem1):
        wid = lax.axis_index("s") * 2 + lax.axis_index("c")
        base = wid * RPW
        bufs = ((pv0, tv0, sem0), (pv1, tv1, sem1))

        def start_chunk(ch):
            pv, tv, sem = bufs[ch % 2]
            ca = pltpu.async_copy(preds_hbm.at[pl.ds(base + ch * CH, CH)], pv, sem)
            cb = pltpu.async_copy(targets_hbm.at[pl.ds(base + ch * CH, CH)], tv, sem)
            return ca, cb

        pending = start_chunk(0)

        lanes = lax.iota(jnp.int32, L)
        tail = lanes >= (L - (C - (NV - 1) * L))
        padv = jnp.full((L,), PAD, jnp.float32)
        zerov = jnp.zeros((L,), jnp.float32)
        onev = jnp.ones((L,), jnp.float32)

        def make_row_step(pv, tv):
            def row_step(r):
                p = [pv[r, pl.ds(i * L, L)] for i in range(NV - 1)]
                p.append(pv[r, pl.ds(C - L, L)])
                k = None
                for i in range(NV):
                    ti = tv[r, pl.ds(i * L, L)] if i < NV - 1 else jnp.where(
                        tail, tv[r, pl.ds(C - L, L)], zerov)
                    plsc.addupdate(tst.at[i], ti)
                    pc = plsc.all_reduce_population_count(ti > 0.5)
                    k = pc if k is None else k + pc
                idx = 8 * L - jnp.clip(k, 1, C)

                s = _sort_top112(p, jnp.where(tail, p[NV - 1], padv), padv)
                v = idx >> 4
                x12 = jnp.where(v <= 1, s[0], s[1])
                x34 = jnp.where(v <= 3, s[2], s[3])
                x56 = jnp.where(v <= 5, s[4], s[5])
                y = jnp.where(v <= 2, x12, x34)
                z = jnp.where(v <= 6, x56, s[6])
                sel = jnp.where(v <= 4, y, z)
                kth = jnp.take_along_axis(
                    sel, idx & (L - 1), axis=0, mode="promise_in_bounds")

                for i in range(NV):
                    ti = tv[r, pl.ds(i * L, L)] if i < NV - 1 else jnp.where(
                        tail, tv[r, pl.ds(C - L, L)], zerov)
                    hard = p[i] >= kth
                    plsc.addupdate(cst.at[i], jnp.where(hard, ti, zerov))

            return row_step

        for i in range(NV):
            cst[i, :] = zerov
            tst[i, :] = zerov
        for ch in range(NCH):
            nxt = start_chunk(ch + 1) if ch + 1 < NCH else None
            pending[0].wait()
            pending[1].wait()
            pv, tv, _ = bufs[ch % 2]
            plsc.parallel_loop(0, CH, unroll=2)(make_row_step(pv, tv))
            pending = nxt

        pltpu.sync_copy(cst, corr_out.at[wid])
        pltpu.sync_copy(tst, tot_out.at[wid])

    return meter


_meter = _make_meter_kernel()


def kernel(preds, targets, corrects, totals):
    cp, tp = _meter(preds, targets)
    cp = cp.sum(axis=0)
    tp = tp.sum(axis=0)
    tail_n = C - (NV - 1) * L
    corr_col = jnp.concatenate([cp[: NV - 1].reshape(-1), cp[NV - 1, L - tail_n :]])
    tot_col = jnp.concatenate([tp[: NV - 1].reshape(-1), tp[NV - 1, L - tail_n :]])
    return (
        corrects + corr_col,
        totals + tot_col,
        corr_col.sum(),
        tot_col.sum(),
    )


# speedup vs baseline: 1.6766x; 1.6766x over previous
import functools

import jax
import jax.numpy as jnp
from jax import lax
from jax.experimental import pallas as pl
from jax.experimental.pallas import tpu as pltpu
from jax.experimental.pallas import tpu_sc as plsc

N = 16384
C = 100
L = 16
NV = 7
NW = 32
RPW = N // NW
CH = 128
NCH = RPW // CH
PAD = -1.0


def _sa(v):
    return plsc.sort_key_val(v, v)[0]


def _sd(v):
    return plsc.sort_key_val(v, v, descending=True)[0]


def _sort_top112(p, w6, padv):
    a0, d1 = _sa(p[0]), _sd(p[1])
    d2, a3 = _sd(p[2]), _sa(p[3])
    d4, a5 = _sd(p[4]), _sa(p[5])
    a6 = _sa(w6)
    lo, hi = jnp.minimum(a0, d1), jnp.maximum(a0, d1)
    b01 = [_sa(lo), _sa(hi)]
    lo, hi = jnp.minimum(d2, a3), jnp.maximum(d2, a3)
    b23 = [_sd(hi), _sd(lo)]
    lo, hi = jnp.minimum(d4, a5), jnp.maximum(d4, a5)
    b45 = [_sd(hi), _sd(lo)]
    l0, l1 = jnp.minimum(b01[0], b23[0]), jnp.minimum(b01[1], b23[1])
    h0, h1 = jnp.maximum(b01[0], b23[0]), jnp.maximum(b01[1], b23[1])
    q0 = [
        _sa(jnp.minimum(l0, l1)), _sa(jnp.maximum(l0, l1)),
        _sa(jnp.minimum(h0, h1)), _sa(jnp.maximum(h0, h1)),
    ]
    m1, mm1 = jnp.maximum(b45[1], a6), jnp.minimum(b45[1], a6)
    q1 = [
        _sd(jnp.maximum(b45[0], m1)), _sd(jnp.minimum(b45[0], m1)),
        _sd(mm1), padv,
    ]
    f = [jnp.minimum(q0[i], q1[i]) for i in range(3)]
    g = [jnp.maximum(q0[i], q1[i]) for i in range(3)] + [q0[3]]
    gm0, gm1 = jnp.minimum(g[0], g[2]), jnp.minimum(g[1], g[3])
    gM0, gM1 = jnp.maximum(g[0], g[2]), jnp.maximum(g[1], g[3])
    s4, s5 = _sa(jnp.minimum(gm0, gm1)), _sa(jnp.maximum(gm0, gm1))
    s6, s7 = _sa(jnp.minimum(gM0, gM1)), _sa(jnp.maximum(gM0, gM1))
    mm, MM = jnp.minimum(f[0], f[2]), jnp.maximum(f[0], f[2])
    s1 = _sa(mm)
    s2, s3 = _sa(jnp.minimum(MM, f[1])), _sa(jnp.maximum(MM, f[1]))
    return [s1, s2, s3, s4, s5, s6, s7]


def _make_meter_kernel():
    mesh = plsc.VectorSubcoreMesh(core_axis_name="c", subcore_axis_name="s")

    @functools.partial(
        pl.kernel,
        out_type=(
            jax.ShapeDtypeStruct((NW, NV * L), jnp.float32),
            jax.ShapeDtypeStruct((NW, NV * L), jnp.float32),
        ),
        mesh=mesh,
        compiler_params=pltpu.CompilerParams(needs_layout_passes=False),
        scratch_types=[
            pltpu.VMEM((CH, C), jnp.float32),
            pltpu.VMEM((CH, C), jnp.float32),
            pltpu.VMEM((CH, C), jnp.float32),
            pltpu.VMEM((CH, C), jnp.float32),
            pltpu.VMEM((NV * L,), jnp.float32),
            pltpu.VMEM((NV * L,), jnp.float32),
            pltpu.SemaphoreType.DMA,
            pltpu.SemaphoreType.DMA,
        ],
    )
    def meter(preds_hbm, targets_hbm, corr_out, tot_out,
              pv0, tv0, pv1, tv1, cst, tst, sem0, sem1):
        wid = lax.axis_index("s") * 2 + lax.axis_index("c")
        base = wid * RPW
        bufs = ((pv0, tv0, sem0), (pv1, tv1, sem1))

        def start_chunk(ch):
            pv, tv, sem = bufs[ch % 2]
            ca = pltpu.async_copy(preds_hbm.at[pl.ds(base + ch * CH, CH)], pv, sem)
            cb = pltpu.async_copy(targets_hbm.at[pl.ds(base + ch * CH, CH)], tv, sem)
            return ca, cb

        pending = start_chunk(0)

        lanes = lax.iota(jnp.int32, L)
        tail = lanes >= (L - (C - (NV - 1) * L))
        padv = jnp.full((L,), PAD, jnp.float32)
        zerov = jnp.zeros((L,), jnp.float32)
        onev = jnp.ones((L,), jnp.float32)

        def make_row_step(pv, tv):
            def row_step(r):
                p = [pv[r, pl.ds(i * L, L)] for i in range(NV - 1)]
                p.append(pv[r, pl.ds(C - L, L)])
                ksum = None
                for i in range(NV):
                    ti = tv[r, pl.ds(i * L, L)] if i < NV - 1 else jnp.where(
                        tail, tv[r, pl.ds(C - L, L)], zerov)
                    off = i * L if i < NV - 1 else C - L
                    plsc.addupdate(tst.at[pl.ds(off, L)], ti)
                    ksum = ti if ksum is None else ksum + ti
                k = jnp.clip(jnp.sum(ksum).astype(jnp.int32), 1, C)
                idx = jnp.full((L,), 8 * L, jnp.int32) - k

                s = _sort_top112(p, jnp.where(tail, p[NV - 1], padv), padv)
                v = idx >> 4
                x12 = jnp.where(v <= 1, s[0], s[1])
                x34 = jnp.where(v <= 3, s[2], s[3])
                x56 = jnp.where(v <= 5, s[4], s[5])
                y = jnp.where(v <= 2, x12, x34)
                z = jnp.where(v <= 6, x56, s[6])
                sel = jnp.where(v <= 4, y, z)
                kth = jnp.take_along_axis(
                    sel, idx & (L - 1), axis=0, mode="promise_in_bounds")

                for i in range(NV):
                    ti = tv[r, pl.ds(i * L, L)] if i < NV - 1 else jnp.where(
                        tail, tv[r, pl.ds(C - L, L)], zerov)
                    off = i * L if i < NV - 1 else C - L
                    hard = p[i] >= kth
                    plsc.addupdate(cst.at[pl.ds(off, L)], jnp.where(hard, ti, zerov))

            return row_step

        for i in range(NV):
            cst[pl.ds(i * L, L)] = zerov
            tst[pl.ds(i * L, L)] = zerov
        for ch in range(NCH):
            nxt = start_chunk(ch + 1) if ch + 1 < NCH else None
            pending[0].wait()
            pending[1].wait()
            pv, tv, _ = bufs[ch % 2]
            plsc.parallel_loop(0, CH, unroll=2)(make_row_step(pv, tv))
            pending = nxt

        pltpu.sync_copy(cst, corr_out.at[wid])
        pltpu.sync_copy(tst, tot_out.at[wid])

    return meter


_meter = _make_meter_kernel()


def kernel(preds, targets, corrects, totals):
    cp, tp = _meter(preds, targets)
    corr_col = cp.sum(axis=0)[:C]
    tot_col = tp.sum(axis=0)[:C]
    return (
        corrects + corr_col,
        totals + tot_col,
        corr_col.sum(),
        tot_col.sum(),
    )


# speedup vs baseline: 1.6978x; 1.0126x over previous
import functools

import jax
import jax.numpy as jnp
from jax import lax
from jax.experimental import pallas as pl
from jax.experimental.pallas import tpu as pltpu
from jax.experimental.pallas import tpu_sc as plsc

N = 16384
C = 100
L = 16
NV = 7
NW = 32
RPW = N // NW
CH = 128
NCH = RPW // CH
PAD = -1.0


def _sa(v):
    return plsc.sort_key_val(v, v)[0]


def _sd(v):
    return plsc.sort_key_val(v, v, descending=True)[0]


def _sort_top112(p, w6, padv):
    a0, d1 = _sa(p[0]), _sd(p[1])
    d2, a3 = _sd(p[2]), _sa(p[3])
    d4, a5 = _sd(p[4]), _sa(p[5])
    a6 = _sa(w6)
    lo, hi = jnp.minimum(a0, d1), jnp.maximum(a0, d1)
    b01 = [_sa(lo), _sa(hi)]
    lo, hi = jnp.minimum(d2, a3), jnp.maximum(d2, a3)
    b23 = [_sd(hi), _sd(lo)]
    lo, hi = jnp.minimum(d4, a5), jnp.maximum(d4, a5)
    b45 = [_sd(hi), _sd(lo)]
    l0, l1 = jnp.minimum(b01[0], b23[0]), jnp.minimum(b01[1], b23[1])
    h0, h1 = jnp.maximum(b01[0], b23[0]), jnp.maximum(b01[1], b23[1])
    q0 = [
        _sa(jnp.minimum(l0, l1)), _sa(jnp.maximum(l0, l1)),
        _sa(jnp.minimum(h0, h1)), _sa(jnp.maximum(h0, h1)),
    ]
    m1, mm1 = jnp.maximum(b45[1], a6), jnp.minimum(b45[1], a6)
    q1 = [
        _sd(jnp.maximum(b45[0], m1)), _sd(jnp.minimum(b45[0], m1)),
        _sd(mm1), padv,
    ]
    f = [jnp.minimum(q0[i], q1[i]) for i in range(3)]
    g = [jnp.maximum(q0[i], q1[i]) for i in range(3)] + [q0[3]]
    gm0, gm1 = jnp.minimum(g[0], g[2]), jnp.minimum(g[1], g[3])
    gM0, gM1 = jnp.maximum(g[0], g[2]), jnp.maximum(g[1], g[3])
    b4, b5 = jnp.minimum(gm0, gm1), jnp.maximum(gm0, gm1)
    b6, b7 = jnp.minimum(gM0, gM1), jnp.maximum(gM0, gM1)
    mm, MM = jnp.minimum(f[0], f[2]), jnp.maximum(f[0], f[2])
    b1 = mm
    b2, b3 = jnp.minimum(MM, f[1]), jnp.maximum(MM, f[1])
    return [b1, b2, b3, b4, b5, b6, b7]


def _make_meter_kernel():
    mesh = plsc.VectorSubcoreMesh(core_axis_name="c", subcore_axis_name="s")

    @functools.partial(
        pl.kernel,
        out_type=(
            jax.ShapeDtypeStruct((NW, NV * L), jnp.float32),
            jax.ShapeDtypeStruct((NW, NV * L), jnp.float32),
        ),
        mesh=mesh,
        compiler_params=pltpu.CompilerParams(needs_layout_passes=False),
        scratch_types=[
            pltpu.VMEM((CH, C), jnp.float32),
            pltpu.VMEM((CH, C), jnp.float32),
            pltpu.VMEM((CH, C), jnp.float32),
            pltpu.VMEM((CH, C), jnp.float32),
            pltpu.VMEM((NV * L,), jnp.float32),
            pltpu.VMEM((NV * L,), jnp.float32),
            pltpu.SemaphoreType.DMA,
            pltpu.SemaphoreType.DMA,
        ],
    )
    def meter(preds_hbm, targets_hbm, corr_out, tot_out,
              pv0, tv0, pv1, tv1, cst, tst, sem0, sem1):
        wid = lax.axis_index("s") * 2 + lax.axis_index("c")
        base = wid * RPW
        bufs = ((pv0, tv0, sem0), (pv1, tv1, sem1))

        def start_chunk(ch):
            pv, tv, sem = bufs[ch % 2]
            ca = pltpu.async_copy(preds_hbm.at[pl.ds(base + ch * CH, CH)], pv, sem)
            cb = pltpu.async_copy(targets_hbm.at[pl.ds(base + ch * CH, CH)], tv, sem)
            return ca, cb

        pending = start_chunk(0)

        lanes = lax.iota(jnp.int32, L)
        tail = lanes >= (L - (C - (NV - 1) * L))
        padv = jnp.full((L,), PAD, jnp.float32)
        zerov = jnp.zeros((L,), jnp.float32)

        def make_row_step(pv, tv):
            def row_step(r):
                p = [pv[r, pl.ds(i * L, L)] for i in range(NV - 1)]
                p.append(pv[r, pl.ds(C - L, L)])
                ksum = None
                for i in range(NV):
                    ti = tv[r, pl.ds(i * L, L)] if i < NV - 1 else jnp.where(
                        tail, tv[r, pl.ds(C - L, L)], zerov)
                    off = i * L if i < NV - 1 else C - L
                    plsc.addupdate(tst.at[pl.ds(off, L)], ti)
                    ksum = ti if ksum is None else ksum + ti
                k = jnp.clip(jnp.sum(ksum).astype(jnp.int32), 1, C)
                idx = jnp.full((L,), 8 * L, jnp.int32) - k

                s = _sort_top112(p, jnp.where(tail, p[NV - 1], padv), padv)
                v = idx >> 4
                x12 = jnp.where(v <= 1, s[0], s[1])
                x34 = jnp.where(v <= 3, s[2], s[3])
                x56 = jnp.where(v <= 5, s[4], s[5])
                y = jnp.where(v <= 2, x12, x34)
                z = jnp.where(v <= 6, x56, s[6])
                kth = jnp.take_along_axis(
                    _sa(jnp.where(v <= 4, y, z)), idx & (L - 1), axis=0,
                    mode="promise_in_bounds")

                for i in range(NV):
                    ti = tv[r, pl.ds(i * L, L)] if i < NV - 1 else jnp.where(
                        tail, tv[r, pl.ds(C - L, L)], zerov)
                    off = i * L if i < NV - 1 else C - L
                    hard = p[i] >= kth
                    plsc.addupdate(cst.at[pl.ds(off, L)], jnp.where(hard, ti, zerov))

            return row_step

        for i in range(NV):
            cst[pl.ds(i * L, L)] = zerov
            tst[pl.ds(i * L, L)] = zerov
        for ch in range(NCH):
            nxt = start_chunk(ch + 1) if ch + 1 < NCH else None
            pending[0].wait()
            pending[1].wait()
            pv, tv, _ = bufs[ch % 2]
            plsc.parallel_loop(0, CH, unroll=2)(make_row_step(pv, tv))
            pending = nxt

        pltpu.sync_copy(cst, corr_out.at[wid])
        pltpu.sync_copy(tst, tot_out.at[wid])

    return meter


_meter = _make_meter_kernel()


def kernel(preds, targets, corrects, totals):
    cp, tp = _meter(preds, targets)
    corr_col = cp.sum(axis=0)[:C]
    tot_col = tp.sum(axis=0)[:C]
    return (
        corrects + corr_col,
        totals + tot_col,
        corr_col.sum(),
        tot_col.sum(),
    )
